# baseline (speedup 1.0000x reference)
import functools

import jax
import jax.numpy as jnp
from jax import lax
from jax.experimental import pallas as pl
from jax.experimental.pallas import tpu as pltpu
from jax.experimental.pallas import tpu_sc as plsc

_NC, _NS = 2, 16
_NW = _NC * _NS

_U_ROWS, _I_ROWS, _DIM = 100000, 1000000, 32
_U_CHUNK = 160
_I_CHUNK = 160
_NBUF = 6


def _phase(src, dst, n_rows, chunk, wid, bufs, sin, sout):
    n_chunks = n_rows // chunk
    j_max = -(-n_chunks // _NW)
    n_groups = -(-j_max // _NBUF)

    def pred(j):
        return (j * _NW + wid) < n_chunks

    def base(j):
        return pl.multiple_of((j * _NW + wid) * chunk, 8)

    def in_copy(j, b):
        return pltpu.make_async_copy(
            src.at[pl.ds(base(j), chunk)], bufs[b].at[pl.ds(0, chunk)], sin[b])

    def out_copy(j, b):
        return pltpu.make_async_copy(
            bufs[b].at[pl.ds(0, chunk)], dst.at[pl.ds(base(j), chunk)], sout[b])

    def direct_copy(j, b):
        return pltpu.make_async_copy(
            src.at[pl.ds(base(j), chunk)], dst.at[pl.ds(base(j), chunk)],
            sin[b])

    def group(g, carry):
        for b in range(_NBUF):
            j = g * _NBUF + b
            pl.when(pred(j))(direct_copy(j, b).start)
        for b in range(_NBUF):
            j = g * _NBUF + b
            pl.when(pred(j))(direct_copy(j, b).wait)
        return carry

    lax.fori_loop(0, n_groups, group, 0)


def _sc_copy_body(u_in, i_in, u_out, i_out, *scratch):
    wid = lax.axis_index("s") * _NC + lax.axis_index("c")
    bufs = scratch[:_NBUF]
    sin = scratch[_NBUF:2 * _NBUF]
    sout = scratch[2 * _NBUF:]
    _phase(i_in, i_out, _I_ROWS, _I_CHUNK, wid, bufs, sin, sout)
    _phase(u_in, u_out, _U_ROWS, _U_CHUNK, wid, bufs, sin, sout)


@functools.partial(
    pl.kernel,
    out_type=(
        jax.ShapeDtypeStruct((_U_ROWS, _DIM), jnp.float32),
        jax.ShapeDtypeStruct((_I_ROWS, _DIM), jnp.float32),
    ),
    mesh=plsc.VectorSubcoreMesh(core_axis_name="c", subcore_axis_name="s"),
    scratch_types=(
        [pltpu.VMEM((_I_CHUNK, _DIM), jnp.float32)] * _NBUF
        + [pltpu.SemaphoreType.DMA] * (2 * _NBUF)
    ),
)
def _sc_copy(u_in, i_in, u_out, i_out, *scratch):
    _sc_copy_body(u_in, i_in, u_out, i_out, *scratch)


def kernel(user_weight, item_weight, edge_index):
    return _sc_copy(user_weight, item_weight)


# speedup vs baseline: 16.9588x; 16.9588x over previous
import functools

import jax
import jax.numpy as jnp
from jax import lax
from jax.experimental import pallas as pl
from jax.experimental.pallas import tpu as pltpu
from jax.experimental.pallas import tpu_sc as plsc

_NC, _NS = 2, 16
_NW = _NC * _NS

_U_ROWS, _I_ROWS, _DIM = 100000, 1000000, 32
_U_CHUNK = 200
_I_CHUNK = 320
_NBUF = 3


def _phase(src, dst, n_rows, chunk, wid, bufs, sin, sout):
    n_chunks = n_rows // chunk
    j_max = -(-n_chunks // _NW)
    n_groups = -(-j_max // _NBUF)

    def pred(j):
        return (j * _NW + wid) < n_chunks

    def base(j):
        return pl.multiple_of((j * _NW + wid) * chunk, 8)

    def in_copy(j, b):
        return pltpu.make_async_copy(
            src.at[pl.ds(base(j), chunk)], bufs[b].at[pl.ds(0, chunk)], sin[b])

    def out_copy(j, b):
        return pltpu.make_async_copy(
            bufs[b].at[pl.ds(0, chunk)], dst.at[pl.ds(base(j), chunk)], sout[b])

    def group(g, carry):
        for b in range(_NBUF):
            j = g * _NBUF + b
            pl.when(pred(j))(in_copy(j, b).start)
        for b in range(_NBUF):
            j = g * _NBUF + b

            @pl.when(pred(j))
            def _():
                in_copy(j, b).wait()
                out_copy(j, b).start()
        for b in range(_NBUF):
            j = g * _NBUF + b
            pl.when(pred(j))(out_copy(j, b).wait)
        return carry

    lax.fori_loop(0, n_groups, group, 0)


def _sc_copy_body(u_in, i_in, u_out, i_out, *scratch):
    wid = lax.axis_index("s") * _NC + lax.axis_index("c")
    bufs = scratch[:_NBUF]
    sin = scratch[_NBUF:2 * _NBUF]
    sout = scratch[2 * _NBUF:]
    _phase(i_in, i_out, _I_ROWS, _I_CHUNK, wid, bufs, sin, sout)
    _phase(u_in, u_out, _U_ROWS, _U_CHUNK, wid, bufs, sin, sout)


@functools.partial(
    pl.kernel,
    out_type=(
        jax.ShapeDtypeStruct((_U_ROWS, _DIM), jnp.float32),
        jax.ShapeDtypeStruct((_I_ROWS, _DIM), jnp.float32),
    ),
    mesh=plsc.VectorSubcoreMesh(core_axis_name="c", subcore_axis_name="s"),
    scratch_types=(
        [pltpu.VMEM((_I_CHUNK, _DIM), jnp.float32)] * _NBUF
        + [pltpu.SemaphoreType.DMA] * (2 * _NBUF)
    ),
)
def _sc_copy(u_in, i_in, u_out, i_out, *scratch):
    _sc_copy_body(u_in, i_in, u_out, i_out, *scratch)


def kernel(user_weight, item_weight, edge_index):
    return _sc_copy(user_weight, item_weight)
